# Initial kernel scaffold; baseline (speedup 1.0000x reference)
import jax
import jax.numpy as jnp
from jax.experimental import pallas as pl


def kernel(x_hat, likelihood_y, points):
    raise NotImplementedError("write your pallas kernel here")


# speedup vs baseline: 1.0072x; 1.0072x over previous
import jax
import jax.numpy as jnp
from jax.experimental import pallas as pl
from jax.experimental.pallas import tpu as pltpu

_N = 4
_P = 4096
_C8 = 8
_IBLK = 512
_IB = _P // _IBLK
_LMBDA = 1.0


def _chamfer_kernel(xt_ref, yt_ref, lik_ref, cham_ref, bits_ref,
                    colmin_ref, acc_ref):
    n = pl.program_id(0)
    i = pl.program_id(1)

    @pl.when(jnp.logical_and(n == 0, i == 0))
    def _():
        acc_ref[1] = 0.0
        acc_ref[2] = 0.0

    @pl.when(i == 0)
    def _():
        acc_ref[0] = 0.0
        colmin_ref[...] = jnp.full_like(colmin_ref[...], jnp.inf)
        acc_ref[2] += jnp.sum(jnp.log2(lik_ref[0]))

    xt = xt_ref[0]
    yt = yt_ref[0]
    x2 = jnp.sum(xt * xt, axis=0)[:, None]
    y2 = jnp.sum(yt * yt, axis=0)[None, :]
    xy = jax.lax.dot_general(
        xt, yt, (((0,), (0,)), ((), ())),
        preferred_element_type=jnp.float32)
    d = jnp.maximum(x2 + y2 - 2.0 * xy, 0.0)

    acc_ref[0] += jnp.sum(jnp.min(d, axis=1))
    colmin_ref[0] = jnp.minimum(colmin_ref[0], jnp.min(d, axis=0))

    @pl.when(i == _IB - 1)
    def _():
        acc_ref[1] += (acc_ref[0] + jnp.sum(colmin_ref[0])) / _P

    @pl.when(jnp.logical_and(n == _N - 1, i == _IB - 1))
    def _():
        cham_ref[0] = jnp.full((8, 128), acc_ref[1], jnp.float32)
        bits_ref[0] = jnp.full((8, 128), acc_ref[2], jnp.float32)


def _run(x_hat, likelihood_y, points, interpret=False):
    xt = jnp.pad(x_hat, ((0, 0), (0, 0), (0, _C8 - 3))).transpose(0, 2, 1)
    yt = jnp.pad(points, ((0, 0), (0, 0), (0, _C8 - 3))).transpose(0, 2, 1)
    lik = likelihood_y.reshape(_N, 64, 128)
    cham, bits = pl.pallas_call(
        _chamfer_kernel,
        grid=(_N, _IB),
        in_specs=[
            pl.BlockSpec((1, _C8, _IBLK), lambda n, i: (n, 0, i)),
            pl.BlockSpec((1, _C8, _P), lambda n, i: (n, 0, 0)),
            pl.BlockSpec((1, 64, 128), lambda n, i: (n, 0, 0)),
        ],
        out_specs=[
            pl.BlockSpec((1, 8, 128), lambda n, i: (0, 0, 0)),
            pl.BlockSpec((1, 8, 128), lambda n, i: (0, 0, 0)),
        ],
        out_shape=[
            jax.ShapeDtypeStruct((1, 8, 128), jnp.float32),
            jax.ShapeDtypeStruct((1, 8, 128), jnp.float32),
        ],
        scratch_shapes=[
            pltpu.VMEM((1, _P), jnp.float32),
            pltpu.SMEM((3,), jnp.float32),
        ],
        interpret=interpret,
    )(xt, yt, lik)

    rec_loss = cham[0, 0, 0] / _N
    bit_y_loss = bits[0, 0, 0] / (-_N)
    bpp_y_loss = bit_y_loss / _P
    bit_loss = bit_y_loss
    bpp_loss = bit_loss / _P
    loss = bpp_loss + _LMBDA * rec_loss
    return (loss, bit_y_loss, bpp_y_loss, bit_loss, bpp_loss, rec_loss)


@jax.jit
def kernel(x_hat, likelihood_y, points):
    return _run(x_hat, likelihood_y, points)


# speedup vs baseline: 1.2334x; 1.2247x over previous
import jax
import jax.numpy as jnp
from jax.experimental import pallas as pl
from jax.experimental.pallas import tpu as pltpu

_N = 4
_P = 4096
_C8 = 8
_IBLK = 512
_IB = _P // _IBLK
_LMBDA = 1.0


def _chamfer_kernel(xts_ref, yt_ref, lik_ref, cham_ref, bits_ref,
                    colmin_ref, acc_ref):
    n = pl.program_id(0)
    i = pl.program_id(1)

    @pl.when(jnp.logical_and(n == 0, i == 0))
    def _():
        acc_ref[1] = 0.0
        acc_ref[2] = 0.0

    @pl.when(i == 0)
    def _():
        acc_ref[0] = 0.0
        colmin_ref[...] = jnp.full_like(colmin_ref[...], jnp.inf)
        acc_ref[2] += jnp.sum(jnp.log2(lik_ref[0]))

    xts = xts_ref[0]
    yt = yt_ref[0]
    x2 = 0.25 * jnp.sum(xts * xts, axis=0)[:, None]
    y2 = jnp.sum(yt * yt, axis=0)[None, :]
    nxy2 = jax.lax.dot_general(
        xts, yt, (((0,), (0,)), ((), ())),
        preferred_element_type=jnp.float32)
    d = (x2 + y2) + nxy2
    acc_ref[0] += jnp.sum(jnp.maximum(jnp.min(d, axis=1), 0.0))
    colmin_ref[0] = jnp.minimum(colmin_ref[0], jnp.min(d, axis=0))

    @pl.when(i == _IB - 1)
    def _():
        acc_ref[1] += (acc_ref[0]
                       + jnp.sum(jnp.maximum(colmin_ref[0], 0.0))) / _P

    @pl.when(jnp.logical_and(n == _N - 1, i == _IB - 1))
    def _():
        cham_ref[0] = jnp.full((8, 128), acc_ref[1], jnp.float32)
        bits_ref[0] = jnp.full((8, 128), acc_ref[2], jnp.float32)


def _run(x_hat, likelihood_y, points, interpret=False):
    xts = (-2.0 * jnp.pad(x_hat, ((0, 0), (0, 0), (0, _C8 - 3)))
           ).transpose(0, 2, 1)
    yt = jnp.pad(points, ((0, 0), (0, 0), (0, _C8 - 3))).transpose(0, 2, 1)
    lik = likelihood_y.reshape(_N, 64, 128)
    cham, bits = pl.pallas_call(
        _chamfer_kernel,
        grid=(_N, _IB),
        in_specs=[
            pl.BlockSpec((1, _C8, _IBLK), lambda n, i: (n, 0, i)),
            pl.BlockSpec((1, _C8, _P), lambda n, i: (n, 0, 0)),
            pl.BlockSpec((1, 64, 128), lambda n, i: (n, 0, 0)),
        ],
        out_specs=[
            pl.BlockSpec((1, 8, 128), lambda n, i: (0, 0, 0)),
            pl.BlockSpec((1, 8, 128), lambda n, i: (0, 0, 0)),
        ],
        out_shape=[
            jax.ShapeDtypeStruct((1, 8, 128), jnp.float32),
            jax.ShapeDtypeStruct((1, 8, 128), jnp.float32),
        ],
        scratch_shapes=[
            pltpu.VMEM((1, _P), jnp.float32),
            pltpu.SMEM((3,), jnp.float32),
        ],
        interpret=interpret,
    )(xts, yt, lik)

    rec_loss = cham[0, 0, 0] / _N
    bit_y_loss = bits[0, 0, 0] / (-_N)
    bpp_y_loss = bit_y_loss / _P
    bit_loss = bit_y_loss
    bpp_loss = bit_loss / _P
    loss = bpp_loss + _LMBDA * rec_loss
    return (loss, bit_y_loss, bpp_y_loss, bit_loss, bpp_loss, rec_loss)


@jax.jit
def kernel(x_hat, likelihood_y, points):
    return _run(x_hat, likelihood_y, points)


# speedup vs baseline: 1.3390x; 1.0856x over previous
import jax
import jax.numpy as jnp
from jax.experimental import pallas as pl
from jax.experimental.pallas import tpu as pltpu

_N = 4
_P = 4096
_C8 = 8
_IBLK = 512
_IB = _P // _IBLK
_LMBDA = 1.0


def _chamfer_kernel(xts_ref, yt_ref, lik_ref, cham_ref, bits_ref,
                    colmin_ref, acc_ref):
    n = pl.program_id(0)
    i = pl.program_id(1)

    @pl.when(jnp.logical_and(n == 0, i == 0))
    def _():
        acc_ref[1] = 0.0
        acc_ref[2] = 0.0

    @pl.when(i == 0)
    def _():
        acc_ref[0] = 0.0
        colmin_ref[...] = jnp.full_like(colmin_ref[...], jnp.inf)
        acc_ref[2] += jnp.sum(jnp.log2(lik_ref[0]))

    xts = xts_ref[0]
    yt = yt_ref[0]
    x2 = 0.25 * jnp.sum(xts * xts, axis=0)
    y2 = jnp.sum(yt * yt, axis=0)
    rx = jax.lax.broadcasted_iota(jnp.int32, (_C8, _IBLK), 0)
    ry = jax.lax.broadcasted_iota(jnp.int32, (_C8, _P), 0)
    xa = jnp.where(rx == 3, x2[None, :],
                   jnp.where(rx == 4, 1.0, xts))
    ya = jnp.where(ry == 3, 1.0,
                   jnp.where(ry == 4, y2[None, :], yt))
    d = jax.lax.dot_general(
        xa, ya, (((0,), (0,)), ((), ())),
        preferred_element_type=jnp.float32)
    acc_ref[0] += jnp.sum(jnp.maximum(jnp.min(d, axis=1), 0.0))
    colmin_ref[0] = jnp.minimum(colmin_ref[0], jnp.min(d, axis=0))

    @pl.when(i == _IB - 1)
    def _():
        acc_ref[1] += (acc_ref[0]
                       + jnp.sum(jnp.maximum(colmin_ref[0], 0.0))) / _P

    @pl.when(jnp.logical_and(n == _N - 1, i == _IB - 1))
    def _():
        cham_ref[0] = jnp.full((8, 128), acc_ref[1], jnp.float32)
        bits_ref[0] = jnp.full((8, 128), acc_ref[2], jnp.float32)


def _run(x_hat, likelihood_y, points, interpret=False):
    xts = (-2.0 * jnp.pad(x_hat, ((0, 0), (0, 0), (0, _C8 - 3)))
           ).transpose(0, 2, 1)
    yt = jnp.pad(points, ((0, 0), (0, 0), (0, _C8 - 3))).transpose(0, 2, 1)
    lik = likelihood_y.reshape(_N, 64, 128)
    cham, bits = pl.pallas_call(
        _chamfer_kernel,
        grid=(_N, _IB),
        in_specs=[
            pl.BlockSpec((1, _C8, _IBLK), lambda n, i: (n, 0, i)),
            pl.BlockSpec((1, _C8, _P), lambda n, i: (n, 0, 0)),
            pl.BlockSpec((1, 64, 128), lambda n, i: (n, 0, 0)),
        ],
        out_specs=[
            pl.BlockSpec((1, 8, 128), lambda n, i: (0, 0, 0)),
            pl.BlockSpec((1, 8, 128), lambda n, i: (0, 0, 0)),
        ],
        out_shape=[
            jax.ShapeDtypeStruct((1, 8, 128), jnp.float32),
            jax.ShapeDtypeStruct((1, 8, 128), jnp.float32),
        ],
        scratch_shapes=[
            pltpu.VMEM((1, _P), jnp.float32),
            pltpu.SMEM((3,), jnp.float32),
        ],
        interpret=interpret,
    )(xts, yt, lik)

    rec_loss = cham[0, 0, 0] / _N
    bit_y_loss = bits[0, 0, 0] / (-_N)
    bpp_y_loss = bit_y_loss / _P
    bit_loss = bit_y_loss
    bpp_loss = bit_loss / _P
    loss = bpp_loss + _LMBDA * rec_loss
    return (loss, bit_y_loss, bpp_y_loss, bit_loss, bpp_loss, rec_loss)


@jax.jit
def kernel(x_hat, likelihood_y, points):
    return _run(x_hat, likelihood_y, points)
